# Initial kernel scaffold; baseline (speedup 1.0000x reference)
import jax
import jax.numpy as jnp
from jax.experimental import pallas as pl


def kernel(write_weights, free_gate, read_weights, prev_usage):
    raise NotImplementedError("write your pallas kernel here")


# speedup vs baseline: 1.2406x; 1.2406x over previous
import jax
import jax.numpy as jnp
from jax.experimental import pallas as pl

B, W, R, M = 256, 4, 8, 8192
BB = 16


def _freeness_kernel(ww_ref, fg_ref, rw_ref, prev_ref, out_ref):
    prev = prev_ref[...]
    keep = jnp.float32(1.0)
    for w in range(W):
        keep = keep * (1.0 - ww_ref[:, w, :])
    usage = prev + (1.0 - prev) * (1.0 - keep)
    phi = jnp.float32(1.0)
    for r in range(R):
        fg_r = fg_ref[:, r][:, None]
        phi = phi * (1.0 - fg_r * rw_ref[:, r, :])
    out_ref[...] = usage * phi


def kernel(write_weights, free_gate, read_weights, prev_usage):
    grid = (B // BB,)
    return pl.pallas_call(
        _freeness_kernel,
        grid=grid,
        in_specs=[
            pl.BlockSpec((BB, W, M), lambda i: (i, 0, 0)),
            pl.BlockSpec((BB, R), lambda i: (i, 0)),
            pl.BlockSpec((BB, R, M), lambda i: (i, 0, 0)),
            pl.BlockSpec((BB, M), lambda i: (i, 0)),
        ],
        out_specs=pl.BlockSpec((BB, M), lambda i: (i, 0)),
        out_shape=jax.ShapeDtypeStruct((B, M), jnp.float32),
    )(write_weights, free_gate, read_weights, prev_usage)
